# Initial kernel scaffold; baseline (speedup 1.0000x reference)
import jax
import jax.numpy as jnp
from jax.experimental import pallas as pl


def kernel(node_features, edge_features, from_idx, to_idx, graph_idx, n_graphs, params):
    raise NotImplementedError("write your pallas kernel here")


# speedup vs baseline: 1.3826x; 1.3826x over previous
import functools

import jax
import jax.numpy as jnp
from jax import lax
from jax.experimental import pallas as pl
from jax.experimental.pallas import tpu as pltpu
from jax.experimental.pallas import tpu_sc as plsc

NN = 10000
NNP = 10240
NE = 320000
MH = 64
NSTATE = 32
NGR = 128
NC, NS = 2, 16
NW = NC * NS
CH = 64
NCHT = 160
NEP = NW * NCHT * CH
EPT = NEP // NW
NPT = NNP // NW
GPT = NGR // NW

_mesh = functools.partial(
    plsc.VectorSubcoreMesh,
    core_axis_name="c", subcore_axis_name="s", num_cores=NC, num_subcores=NS)

_f32 = jnp.float32
_i32 = jnp.int32


def _widx():
    c = lax.axis_index("c")
    s = lax.axis_index("s")
    return c, s, c * NS + s


def _zero_rows(zb, ncols):
    z16 = jnp.zeros((16,), _f32)

    def row(i, carry):
        for j in range(ncols // 16):
            zb[i, pl.ds(j * 16, 16)] = z16
        return carry

    lax.fori_loop(0, zb.shape[0], row, 0)


def _lane(vec16, lane):
    return jnp.sum(jnp.where(lax.iota(_i32, 16) == lane, vec16, 0))


def _make_edge_kernel():
    @functools.partial(
        pl.kernel,
        out_type=jax.ShapeDtypeStruct((NEP, 128), _f32),
        mesh=_mesh(),
        scratch_types=[
            pltpu.VMEM((8, CH), _i32),
            pltpu.VMEM((8, CH), _i32),
            pltpu.VMEM((CH, 128), _f32),
            pltpu.VMEM((CH, 128), _f32),
            pltpu.VMEM((CH, 128), _f32),
            pltpu.VMEM((CH, 128), _f32),
            pltpu.SemaphoreType.DMA,
            pltpu.SemaphoreType.DMA,
        ],
    )
    def edge_kernel(ad_hbm, bc_hbm, e_hbm, fidx_hbm, tidx_hbm, ho,
                    fv, tv, gad, gbc, ev, hb, sem1, sem2):
        c, s, wid = _widx()

        def group(j, carry):
            pltpu.sync_copy(fidx_hbm.at[wid, pl.ds(j * 8, 8)], fv)
            pltpu.sync_copy(tidx_hbm.at[wid, pl.ds(j * 8, 8)], tv)

            def chunk(k, kc):
                base = wid * EPT + j * (8 * CH) + k * CH
                cp1 = pltpu.async_copy(ad_hbm.at[fv.at[k]], gad, sem1)
                cp2 = pltpu.async_copy(bc_hbm.at[tv.at[k]], gbc, sem2)
                pltpu.sync_copy(e_hbm.at[pl.ds(base, CH)], ev)
                cp1.wait()
                cp2.wait()

                def row(r, rc):
                    for jj in range(128 // 16):
                        o = jj * 16
                        hb[r, pl.ds(o, 16)] = jnp.maximum(
                            gad[r, pl.ds(o, 16)] + gbc[r, pl.ds(o, 16)]
                            + ev[r, pl.ds(o, 16)], 0.0)
                    return rc

                lax.fori_loop(0, CH, row, 0)
                pltpu.sync_copy(hb, ho.at[pl.ds(base, CH)])
                return kc

            lax.fori_loop(0, 8, chunk, 0)
            return carry

        lax.fori_loop(0, NCHT // 8, group, 0)

    return edge_kernel


def _dyng(vec, idx):
    dn = lax.GatherDimensionNumbers(
        offset_dims=(), collapsed_slice_dims=(0,), start_index_map=(0,))
    return lax.gather(vec, idx.reshape(16, 1), dn, (1,),
                      mode=lax.GatherScatterMode.PROMISE_IN_BOUNDS)


NCHB = EPT // 64


def _seg_scan_chunk(dv, hg, pc, prevv, accv, iota16, shifti, ncols, k):
    nsl = ncols // 16
    for q in range(4):
        dq = dv[k, pl.ds(q * 16, 16)]
        if q == 0:
            pl16 = prevv[pl.ds(0, 16)]
        else:
            pl16 = _dyng(dv[k, pl.ds((q - 1) * 16, 16)],
                         jnp.full((16,), 15, _i32))
        sh = _dyng(dq, shifti)
        sh = jnp.where(iota16 == 0, pl16, sh)
        eqf = jnp.where(dq == sh, 1.0, 0.0).astype(_f32)
        for r in range(16):
            flag = _dyng(eqf, jnp.full((16,), r, _i32))
            row = q * 16 + r
            for j in range(nsl):
                a = accv[j, pl.ds(0, 16)] * flag + hg[row, pl.ds(j * 16, 16)]
                accv[j, pl.ds(0, 16)] = a
                pc[row, pl.ds(j * 16, 16)] = a
        prevv[pl.ds(0, 16)] = _dyng(dq, jnp.full((16,), 15, _i32))


def _make_scan_kernel():
    @functools.partial(
        pl.kernel,
        out_type=(jax.ShapeDtypeStruct((NEP, 128), _f32),
                  jax.ShapeDtypeStruct((NEP, 128), _f32)),
        mesh=_mesh(),
        scratch_types=[
            pltpu.VMEM((8, 64), _i32),
            pltpu.VMEM((8, 64), _i32),
            pltpu.VMEM((64, 128), _f32),
            pltpu.VMEM((64, 128), _f32),
            pltpu.VMEM((16,), _i32),
            pltpu.VMEM((4, 16), _f32),
            pltpu.SemaphoreType.DMA,
        ],
    )
    def scan_kernel(h_hbm, ord1_hbm, ord2_hbm, d1_hbm, d2_hbm,
                    p1o, p2o, ordv, dv, hg, pc, prevv, accv, sem):
        c, s, wid = _widx()
        iota16 = lax.iota(_i32, 16)
        shifti = jnp.maximum(iota16 - 1, 0)
        z16 = jnp.zeros((16,), _f32)

        for (coff, ord_hbm, d_hbm, po) in (
                (0, ord1_hbm, d1_hbm, p1o),
                (MH, ord2_hbm, d2_hbm, p2o)):
            prevv[pl.ds(0, 16)] = jnp.full((16,), -1, _i32)
            for j in range(4):
                accv[j, pl.ds(0, 16)] = z16

            def group(g, carry):
                row0 = wid * NCHB + g * 8
                pltpu.sync_copy(ord_hbm.at[pl.ds(row0, 8)], ordv)
                pltpu.sync_copy(d_hbm.at[pl.ds(row0, 8)], dv)

                def chunk(k, kc):
                    off = wid * EPT + g * 512 + k * 64
                    pltpu.async_copy(h_hbm.at[ordv.at[k]], hg, sem).wait()
                    for q in range(4):
                        dq = dv[k, pl.ds(q * 16, 16)]
                        if q == 0:
                            pl16 = prevv[pl.ds(0, 16)]
                        else:
                            pl16 = _dyng(dv[k, pl.ds((q - 1) * 16, 16)],
                                         jnp.full((16,), 15, _i32))
                        sh = _dyng(dq, shifti)
                        sh = jnp.where(iota16 == 0, pl16, sh)
                        eqf = jnp.where(dq == sh, 1.0, 0.0).astype(_f32)
                        for r in range(16):
                            flag = _dyng(eqf, jnp.full((16,), r, _i32))
                            row = q * 16 + r
                            for j in range(4):
                                a = (accv[j, pl.ds(0, 16)] * flag
                                     + hg[row, pl.ds(coff + j * 16, 16)])
                                accv[j, pl.ds(0, 16)] = a
                                pc[row, pl.ds(j * 16, 16)] = a
                        prevv[pl.ds(0, 16)] = _dyng(
                            dq, jnp.full((16,), 15, _i32))
                    pltpu.sync_copy(pc, po.at[pl.ds(off, 64)])
                    return kc

                lax.fori_loop(0, 8, chunk, 0)
                return carry

            lax.fori_loop(0, NCHB // 8, group, 0)

    return scan_kernel


def _make_pick_kernel():
    @functools.partial(
        pl.kernel,
        out_type=(jax.ShapeDtypeStruct((NNP, 128), _f32),
                  jax.ShapeDtypeStruct((NNP, 128), _f32)),
        mesh=_mesh(),
        scratch_types=[
            pltpu.VMEM((5, 64), _i32),
            pltpu.VMEM((5, 64), _i32),
            pltpu.VMEM((64, 128), _f32),
            pltpu.SemaphoreType.DMA,
        ],
    )
    def pick_kernel(p1_hbm, p2_hbm, e1_hbm, e2_hbm, o1, o2, i1, i2, hg, sem):
        c, s, wid = _widx()
        pltpu.sync_copy(e1_hbm.at[wid], i1)
        pltpu.sync_copy(e2_hbm.at[wid], i2)
        for g in range(5):
            pltpu.async_copy(p1_hbm.at[i1.at[g]], hg, sem).wait()
            pltpu.sync_copy(hg, o1.at[pl.ds(wid * NPT + g * 64, 64)])
        for g in range(5):
            pltpu.async_copy(p2_hbm.at[i2.at[g]], hg, sem).wait()
            pltpu.sync_copy(hg, o2.at[pl.ds(wid * NPT + g * 64, 64)])

    return pick_kernel


def _make_pool_kernel():
    NCHP = NNP // NW // 64

    @functools.partial(
        pl.kernel,
        out_type=jax.ShapeDtypeStruct((NNP, NGR), _f32),
        mesh=_mesh(),
        scratch_types=[
            pltpu.VMEM((NCHP * 64,), _i32),
            pltpu.VMEM((64, NGR), _f32),
            pltpu.VMEM((64, NGR), _f32),
            pltpu.VMEM((16,), _i32),
            pltpu.VMEM((8, 16), _f32),
        ],
    )
    def pool_kernel(gated_hbm, gidx_hbm, pg, dv3, gv, pc, prevv, accv):
        c, s, wid = _widx()
        iota16 = lax.iota(_i32, 16)
        shifti = jnp.maximum(iota16 - 1, 0)
        z16 = jnp.zeros((16,), _f32)
        base = wid * (NCHP * 64)
        pltpu.sync_copy(gidx_hbm.at[pl.ds(base, NCHP * 64)], dv3)
        prevv[pl.ds(0, 16)] = jnp.full((16,), -1, _i32)
        for j in range(8):
            accv[j, pl.ds(0, 16)] = z16

        def chunk(k, carry):
            off = base + k * 64
            pltpu.sync_copy(gated_hbm.at[pl.ds(off, 64)], gv)
            for q in range(4):
                dq = dv3[pl.ds(k * 64 + q * 16, 16)]
                if q == 0:
                    pl16 = prevv[pl.ds(0, 16)]
                else:
                    pl16 = _dyng(dv3[pl.ds(k * 64 + (q - 1) * 16, 16)],
                                 jnp.full((16,), 15, _i32))
                sh = _dyng(dq, shifti)
                sh = jnp.where(iota16 == 0, pl16, sh)
                eqf = jnp.where(dq == sh, 1.0, 0.0).astype(_f32)
                for r in range(16):
                    flag = _dyng(eqf, jnp.full((16,), r, _i32))
                    row = q * 16 + r
                    for j in range(8):
                        a = (accv[j, pl.ds(0, 16)] * flag
                             + gv[row, pl.ds(j * 16, 16)])
                        accv[j, pl.ds(0, 16)] = a
                        pc[row, pl.ds(j * 16, 16)] = a
                prevv[pl.ds(0, 16)] = _dyng(dq, jnp.full((16,), 15, _i32))
            pltpu.sync_copy(pc, pg.at[pl.ds(off, 64)])
            return carry

        lax.fori_loop(0, NCHP, chunk, 0)

    return pool_kernel


def _make_gpick_kernel():
    @functools.partial(
        pl.kernel,
        out_type=jax.ShapeDtypeStruct((NGR, NGR), _f32),
        mesh=_mesh(),
        scratch_types=[
            pltpu.VMEM((1, 64), _i32),
            pltpu.VMEM((64, NGR), _f32),
            pltpu.SemaphoreType.DMA,
        ],
    )
    def gpick_kernel(pg_hbm, ge_hbm, out, iv, hg, sem):
        c, s, wid = _widx()

        @pl.when(wid < 2)
        def _():
            pltpu.sync_copy(ge_hbm.at[wid], iv)
            pltpu.async_copy(pg_hbm.at[iv.at[0]], hg, sem).wait()
            pltpu.sync_copy(hg, out.at[pl.ds(wid * 64, 64)])

    return gpick_kernel


def _full(shape):
    return pl.BlockSpec(shape, lambda *_: (0,) * len(shape))


def _enc_body(nf, wn, bn, wad, wbc, xo, ado, bco):
    x = jnp.dot(nf[...], wn[...], preferred_element_type=_f32,
            precision=jax.lax.Precision.HIGHEST) + bn[...]
    xo[...] = x
    ado[...] = jnp.dot(x, wad[...], preferred_element_type=_f32,
            precision=jax.lax.Precision.HIGHEST)
    bco[...] = jnp.dot(x, wbc[...], preferred_element_type=_f32,
            precision=jax.lax.Precision.HIGHEST)


def _enc_call(nf, wn, bn, wad, wbc):
    blk = 640
    return pl.pallas_call(
        _enc_body,
        grid=(NNP // blk,),
        in_specs=[pl.BlockSpec((blk, 128), lambda i: (i, 0)),
                  _full((128, NSTATE)), _full((1, NSTATE)),
                  _full((NSTATE, 128)), _full((NSTATE, 128))],
        out_specs=[pl.BlockSpec((blk, NSTATE), lambda i: (i, 0)),
                   pl.BlockSpec((blk, 128), lambda i: (i, 0)),
                   pl.BlockSpec((blk, 128), lambda i: (i, 0))],
        out_shape=[jax.ShapeDtypeStruct((NNP, NSTATE), _f32),
                   jax.ShapeDtypeStruct((NNP, 128), _f32),
                   jax.ShapeDtypeStruct((NNP, 128), _f32)],
    )(nf, wn, bn, wad, wbc)


def _ec_body(ef, w, b, eo):
    eo[...] = jnp.dot(ef[...], w[...], preferred_element_type=_f32,
            precision=jax.lax.Precision.HIGHEST) + b[...]


def _ec_call(ef, w, b):
    blk = 10240
    return pl.pallas_call(
        _ec_body,
        grid=(NEP // blk,),
        in_specs=[pl.BlockSpec((blk, 16), lambda i: (i, 0)),
                  _full((16, 128)), _full((1, 128))],
        out_specs=pl.BlockSpec((blk, 128), lambda i: (i, 0)),
        out_shape=jax.ShapeDtypeStruct((NEP, 128), _f32),
    )(ef, w, b)


def _layer_body(s1, s2, pb1, pb2, bd1, bd2, x, cnt1, cnt2, w2, v2, b2, c2,
                u1a, u1b, u1, u2w, u2b, wad, wbc, xo, ado, bco):
    blk = s1.shape[0]
    rows = (pl.program_id(0) * blk
            + lax.broadcasted_iota(_i32, (blk, 1), 0))
    oh1 = (rows == bd1[...]).astype(_f32)
    oh2 = (rows == bd2[...]).astype(_f32)
    s1e = ((s1[...][:, :MH]
            + jnp.dot(oh1, pb1[...][:, :MH], preferred_element_type=_f32,
            precision=jax.lax.Precision.HIGHEST))
           * (cnt1[...] > 0).astype(_f32))
    s2e = ((s2[...][:, :MH]
            + jnp.dot(oh2, pb2[...][:, :MH], preferred_element_type=_f32,
            precision=jax.lax.Precision.HIGHEST))
           * (cnt2[...] > 0).astype(_f32))
    u = (jnp.dot(s1e, w2[...], preferred_element_type=_f32,
            precision=jax.lax.Precision.HIGHEST)
         + jnp.dot(s2e, v2[...], preferred_element_type=_f32,
            precision=jax.lax.Precision.HIGHEST)
         + cnt1[...] * b2[...] + cnt2[...] * c2[...])
    xv = x[...]
    t = jnp.maximum(jnp.dot(u, u1a[...], preferred_element_type=_f32,
            precision=jax.lax.Precision.HIGHEST)
                    + jnp.dot(xv, u1b[...], preferred_element_type=_f32,
            precision=jax.lax.Precision.HIGHEST)
                    + u1[...], 0.0)
    nx = xv + jnp.dot(t, u2w[...], preferred_element_type=_f32,
            precision=jax.lax.Precision.HIGHEST) + u2b[...]
    xo[...] = nx
    ado[...] = jnp.dot(nx, wad[...], preferred_element_type=_f32,
            precision=jax.lax.Precision.HIGHEST)
    bco[...] = jnp.dot(nx, wbc[...], preferred_element_type=_f32,
            precision=jax.lax.Precision.HIGHEST)


def _layer_call(s1, s2, pb1, pb2, bd1, bd2, x, cnt1, cnt2, w2, v2, b2, c2,
                u1a, u1b, u1, u2w, u2b, wad, wbc):
    blk = 640
    return pl.pallas_call(
        _layer_body,
        grid=(NNP // blk,),
        in_specs=[pl.BlockSpec((blk, 128), lambda i: (i, 0)),
                  pl.BlockSpec((blk, 128), lambda i: (i, 0)),
                  _full((NW, 128)), _full((NW, 128)),
                  _full((1, NW)), _full((1, NW)),
                  pl.BlockSpec((blk, NSTATE), lambda i: (i, 0)),
                  pl.BlockSpec((blk, 1), lambda i: (i, 0)),
                  pl.BlockSpec((blk, 1), lambda i: (i, 0)),
                  _full((MH, MH)), _full((MH, MH)),
                  _full((1, MH)), _full((1, MH)),
                  _full((MH, MH)), _full((NSTATE, MH)), _full((1, MH)),
                  _full((MH, NSTATE)), _full((1, NSTATE)),
                  _full((NSTATE, 128)), _full((NSTATE, 128))],
        out_specs=[pl.BlockSpec((blk, NSTATE), lambda i: (i, 0)),
                   pl.BlockSpec((blk, 128), lambda i: (i, 0)),
                   pl.BlockSpec((blk, 128), lambda i: (i, 0))],
        out_shape=[jax.ShapeDtypeStruct((NNP, NSTATE), _f32),
                   jax.ShapeDtypeStruct((NNP, 128), _f32),
                   jax.ShapeDtypeStruct((NNP, 128), _f32)],
    )(s1, s2, pb1, pb2, bd1, bd2, x, cnt1, cnt2, w2, v2, b2, c2,
      u1a, u1b, u1, u2w, u2b, wad, wbc)


def _gate_body(x, a1, b1, go):
    g = jnp.dot(x[...], a1[...], preferred_element_type=_f32,
            precision=jax.lax.Precision.HIGHEST) + b1[...]
    go[...] = jax.nn.sigmoid(g[:, :NGR]) * g[:, NGR:]


def _gate_call(x, a1, b1):
    blk = 640
    return pl.pallas_call(
        _gate_body,
        grid=(NNP // blk,),
        in_specs=[pl.BlockSpec((blk, NSTATE), lambda i: (i, 0)),
                  _full((NSTATE, 2 * NGR)), _full((1, 2 * NGR))],
        out_specs=pl.BlockSpec((blk, NGR), lambda i: (i, 0)),
        out_shape=jax.ShapeDtypeStruct((NNP, NGR), _f32),
    )(x, a1, b1)


def _final_body(gsr, gpb, gbd, gmask, a2, b2, pe, po, do):
    rows = lax.broadcasted_iota(_i32, (NGR, 1), 0)
    oh = (rows == gbd[...]).astype(_f32)
    gs = ((gsr[...] + jnp.dot(oh, gpb[...], preferred_element_type=_f32,
            precision=jax.lax.Precision.HIGHEST))
          * gmask[...])
    gv = jnp.dot(gs, a2[...], preferred_element_type=_f32,
            precision=jax.lax.Precision.HIGHEST) + b2[...]
    diff = (jnp.dot(pe[...], gv, preferred_element_type=_f32,
            precision=jax.lax.Precision.HIGHEST)
            - jnp.dot(po[...], gv, preferred_element_type=_f32,
            precision=jax.lax.Precision.HIGHEST))
    do[...] = jnp.sum(jnp.maximum(diff, 0.0), axis=1, keepdims=True)


def _final_call(gsr, gpb, gbd, gmask, a2, b2, pe, po):
    return pl.pallas_call(
        _final_body,
        grid=(1,),
        in_specs=[_full((NGR, NGR)), _full((NW, NGR)), _full((1, NW)),
                  _full((NGR, 1)),
                  _full((NGR, NGR)), _full((1, NGR)),
                  _full((64, NGR)), _full((64, NGR))],
        out_specs=_full((64, 1)),
        out_shape=jax.ShapeDtypeStruct((64, 1), _f32),
    )(gsr, gpb, gbd, gmask, a2, b2, pe, po)


def _csr(dest_padded):
    order = jnp.argsort(dest_padded).astype(_i32)
    dsorted = dest_padded[order].astype(_i32)
    node_end = jnp.searchsorted(
        dsorted, jnp.arange(1, NNP + 1, dtype=_i32)).astype(_i32)
    cnt = jnp.diff(jnp.concatenate(
        [jnp.zeros((1,), _i32), node_end])).astype(_f32).reshape(NNP, 1)
    endm1 = jnp.maximum(node_end - 1, 0).reshape(NW, NPT // 64, 64)
    b = jnp.arange(1, NW, dtype=_i32) * EPT - 1
    cont = dsorted[b] == dsorted[b + 1]
    bdest = jnp.where(cont, dsorted[b], NNP).astype(_i32)
    bdest = jnp.concatenate([bdest, jnp.full((1,), NNP, _i32)]).reshape(1, NW)
    return (order.reshape(NEP // 64, 64), dsorted.reshape(NEP // 64, 64),
            endm1, cnt, bdest)


def kernel(node_features, edge_features, from_idx, to_idx, graph_idx,
           n_graphs, params):
    with jax.default_matmul_precision("highest"):
        return _forward_impl(node_features, edge_features, from_idx, to_idx,
                             graph_idx, n_graphs, params)


def _forward_impl(node_features, edge_features, from_idx, to_idx, graph_idx,
                  n_graphs, params):
    p = params
    wn, bn = p['enc_node']
    we, be = p['enc_edge']
    w1, b1, w2, b2 = p['msg']
    v1, c1, v2, c2 = p['rmsg']
    uw1, ub1, uw2, ub2 = p['node_upd']
    a1w, a1b = p['agg1']
    a2w, a2b = p['agg2']

    wc = jnp.concatenate([w1[2 * NSTATE:], v1[2 * NSTATE:]], axis=1)
    wec = we @ wc
    bec = (be @ wc + jnp.concatenate([b1, c1])).reshape(1, 128)
    wad = jnp.concatenate([w1[:NSTATE], v1[NSTATE:2 * NSTATE]], axis=1)
    wbc = jnp.concatenate([w1[NSTATE:2 * NSTATE], v1[:NSTATE]], axis=1)
    u1a, u1b = uw1[:MH], uw1[MH:]

    pad_e = NEP - NE
    fpad = jnp.concatenate([from_idx, jnp.full((pad_e,), NN, _i32)])
    tpad = jnp.concatenate([to_idx, jnp.full((pad_e,), NN, _i32)])
    fidx3 = fpad.reshape(NW, NCHT, CH)
    tidx3 = tpad.reshape(NW, NCHT, CH)

    ord_t, dst_t, end_t, cnt_to, bd_t = _csr(tpad)
    ord_f, dst_f, end_f, cnt_from, bd_f = _csr(fpad)

    edge_k = _make_edge_kernel()
    scan_k = _make_scan_kernel()
    pick_k = _make_pick_kernel()
    pool_k = _make_pool_kernel()
    gpick_k = _make_gpick_kernel()

    nf_p = jnp.concatenate(
        [node_features, jnp.zeros((NNP - NN, 128), _f32)], axis=0)
    x, ad, bc = _enc_call(nf_p, wn, bn.reshape(1, -1), wad, wbc)
    ef_p = jnp.concatenate(
        [edge_features, jnp.zeros((NEP - NE, 16), _f32)], axis=0)
    e_const = _ec_call(ef_p, wec, bec)

    for _ in range(5):
        h = edge_k(ad, bc, e_const, fidx3, tidx3)
        p1, p2 = scan_k(h, ord_t, ord_f, dst_t, dst_f)
        s1, s2 = pick_k(p1, p2, end_t, end_f)
        pb1 = p1.reshape(NW, EPT, 128)[:, -1, :]
        pb2 = p2.reshape(NW, EPT, 128)[:, -1, :]
        x, ad, bc = _layer_call(
            s1, s2, pb1, pb2, bd_t, bd_f, x, cnt_to, cnt_from, w2, v2,
            b2.reshape(1, -1), c2.reshape(1, -1), u1a, u1b,
            ub1.reshape(1, -1), uw2, ub2.reshape(1, -1), wad, wbc)

    gated = _gate_call(x, a1w, a1b.reshape(1, -1))
    gidx = jnp.minimum(graph_idx, n_graphs - 1).astype(_i32)
    gidx_p = jnp.concatenate([gidx, jnp.full((NNP - NN,), NGR, _i32)])
    gend = jnp.searchsorted(
        gidx_p, jnp.arange(1, NGR + 1, dtype=_i32)).astype(_i32)
    gcnt = jnp.diff(jnp.concatenate([jnp.zeros((1,), _i32), gend]))
    gmask = (gcnt > 0).astype(_f32).reshape(NGR, 1)
    gendm1 = jnp.maximum(gend - 1, 0).reshape(2, 1, 64)
    gendm1 = jnp.concatenate(
        [gendm1, jnp.zeros((NW - 2, 1, 64), _i32)], axis=0)
    bg = jnp.arange(1, NW, dtype=_i32) * NPT - 1
    gsd = gidx_p
    gcont = gsd[bg] == gsd[bg + 1]
    gbd = jnp.where(gcont, gsd[bg], NGR).astype(_i32)
    gbd = jnp.concatenate([gbd, jnp.full((1,), NGR, _i32)]).reshape(1, NW)

    pg = pool_k(gated, gidx_p)
    gsr = gpick_k(pg, gendm1)
    gpb = pg.reshape(NW, NPT, NGR)[:, -1, :]

    pe = jnp.zeros((64, NGR), _f32).at[jnp.arange(64), 2 * jnp.arange(64)].set(1.0)
    po = jnp.zeros((64, NGR), _f32).at[jnp.arange(64), 2 * jnp.arange(64) + 1].set(1.0)
    d = _final_call(gsr, gpb, gbd, gmask, a2w, a2b.reshape(1, -1), pe, po)
    return d.reshape(64)


# speedup vs baseline: 1.4326x; 1.0362x over previous
import functools

import jax
import jax.numpy as jnp
from jax import lax
from jax.experimental import pallas as pl
from jax.experimental.pallas import tpu as pltpu
from jax.experimental.pallas import tpu_sc as plsc

NN = 10000
NNP = 10240
NE = 320000
MH = 64
NSTATE = 32
NGR = 128
NC, NS = 2, 16
NW = NC * NS
CH = 64
NCHT = 160
NEP = NW * NCHT * CH
EPT = NEP // NW
NPT = NNP // NW
GPT = NGR // NW

_mesh = functools.partial(
    plsc.VectorSubcoreMesh,
    core_axis_name="c", subcore_axis_name="s", num_cores=NC, num_subcores=NS)

_f32 = jnp.float32
_i32 = jnp.int32


def _widx():
    c = lax.axis_index("c")
    s = lax.axis_index("s")
    return c, s, c * NS + s


def _zero_rows(zb, ncols):
    z16 = jnp.zeros((16,), _f32)

    def row(i, carry):
        for j in range(ncols // 16):
            zb[i, pl.ds(j * 16, 16)] = z16
        return carry

    lax.fori_loop(0, zb.shape[0], row, 0)


def _lane(vec16, lane):
    return jnp.sum(jnp.where(lax.iota(_i32, 16) == lane, vec16, 0))


def _make_edge_kernel():
    @functools.partial(
        pl.kernel,
        out_type=jax.ShapeDtypeStruct((NEP, 128), _f32),
        mesh=_mesh(),
        scratch_types=[
            pltpu.VMEM((8, CH), _i32),
            pltpu.VMEM((8, CH), _i32),
            pltpu.VMEM((CH, 128), _f32),
            pltpu.VMEM((CH, 128), _f32),
            pltpu.VMEM((CH, 128), _f32),
            pltpu.VMEM((CH, 128), _f32),
            pltpu.SemaphoreType.DMA,
            pltpu.SemaphoreType.DMA,
        ],
    )
    def edge_kernel(ad_hbm, bc_hbm, e_hbm, fidx_hbm, tidx_hbm, ho,
                    fv, tv, gad, gbc, ev, hb, sem1, sem2):
        c, s, wid = _widx()

        def group(j, carry):
            pltpu.sync_copy(fidx_hbm.at[wid, pl.ds(j * 8, 8)], fv)
            pltpu.sync_copy(tidx_hbm.at[wid, pl.ds(j * 8, 8)], tv)

            def chunk(k, kc):
                base = wid * EPT + j * (8 * CH) + k * CH
                cp1 = pltpu.async_copy(ad_hbm.at[fv.at[k]], gad, sem1)
                cp2 = pltpu.async_copy(bc_hbm.at[tv.at[k]], gbc, sem2)
                pltpu.sync_copy(e_hbm.at[pl.ds(base, CH)], ev)
                cp1.wait()
                cp2.wait()

                def row(r, rc):
                    for jj in range(128 // 16):
                        o = jj * 16
                        hb[r, pl.ds(o, 16)] = jnp.maximum(
                            gad[r, pl.ds(o, 16)] + gbc[r, pl.ds(o, 16)]
                            + ev[r, pl.ds(o, 16)], 0.0)
                    return rc

                lax.fori_loop(0, CH, row, 0)
                pltpu.sync_copy(hb, ho.at[pl.ds(base, CH)])
                return kc

            lax.fori_loop(0, 8, chunk, 0)
            return carry

        lax.fori_loop(0, NCHT // 8, group, 0)

    return edge_kernel


def _dyng(vec, idx):
    dn = lax.GatherDimensionNumbers(
        offset_dims=(), collapsed_slice_dims=(0,), start_index_map=(0,))
    return lax.gather(vec, idx.reshape(16, 1), dn, (1,),
                      mode=lax.GatherScatterMode.PROMISE_IN_BOUNDS)


NCHB = EPT // 64


def _seg_scan_chunk(dv, hg, pc, prevv, accv, iota16, shifti, ncols, k):
    nsl = ncols // 16
    for q in range(4):
        dq = dv[k, pl.ds(q * 16, 16)]
        if q == 0:
            pl16 = prevv[pl.ds(0, 16)]
        else:
            pl16 = _dyng(dv[k, pl.ds((q - 1) * 16, 16)],
                         jnp.full((16,), 15, _i32))
        sh = _dyng(dq, shifti)
        sh = jnp.where(iota16 == 0, pl16, sh)
        eqf = jnp.where(dq == sh, 1.0, 0.0).astype(_f32)
        for r in range(16):
            flag = _dyng(eqf, jnp.full((16,), r, _i32))
            row = q * 16 + r
            for j in range(nsl):
                a = accv[j, pl.ds(0, 16)] * flag + hg[row, pl.ds(j * 16, 16)]
                accv[j, pl.ds(0, 16)] = a
                pc[row, pl.ds(j * 16, 16)] = a
        prevv[pl.ds(0, 16)] = _dyng(dq, jnp.full((16,), 15, _i32))


def _make_scan_kernel():
    @functools.partial(
        pl.kernel,
        out_type=(jax.ShapeDtypeStruct((NEP, 128), _f32),
                  jax.ShapeDtypeStruct((NEP, 128), _f32)),
        mesh=_mesh(),
        scratch_types=[
            pltpu.VMEM((8, 64), _i32),
            pltpu.VMEM((8, 64), _i32),
            pltpu.VMEM((64, 128), _f32),
            pltpu.VMEM((64, 128), _f32),
            pltpu.VMEM((16,), _i32),
            pltpu.VMEM((4, 16), _f32),
            pltpu.SemaphoreType.DMA,
        ],
    )
    def scan_kernel(h_hbm, ord1_hbm, ord2_hbm, d1_hbm, d2_hbm,
                    p1o, p2o, ordv, dv, hg, pc, prevv, accv, sem):
        c, s, wid = _widx()
        iota16 = lax.iota(_i32, 16)
        shifti = jnp.maximum(iota16 - 1, 0)
        z16 = jnp.zeros((16,), _f32)

        for (coff, ord_hbm, d_hbm, po) in (
                (0, ord1_hbm, d1_hbm, p1o),
                (MH, ord2_hbm, d2_hbm, p2o)):
            prevv[pl.ds(0, 16)] = jnp.full((16,), -1, _i32)
            for j in range(4):
                accv[j, pl.ds(0, 16)] = z16

            def group(g, carry):
                row0 = wid * NCHB + g * 8
                pltpu.sync_copy(ord_hbm.at[pl.ds(row0, 8)], ordv)
                pltpu.sync_copy(d_hbm.at[pl.ds(row0, 8)], dv)

                def chunk(k, kc):
                    off = wid * EPT + g * 512 + k * 64
                    pltpu.async_copy(h_hbm.at[ordv.at[k]], hg, sem).wait()
                    acc = [accv[j, pl.ds(0, 16)] for j in range(4)]
                    for q in range(4):
                        dq = dv[k, pl.ds(q * 16, 16)]
                        if q == 0:
                            pl16 = prevv[pl.ds(0, 16)]
                        else:
                            pl16 = _dyng(dv[k, pl.ds((q - 1) * 16, 16)],
                                         jnp.full((16,), 15, _i32))
                        sh = _dyng(dq, shifti)
                        sh = jnp.where(iota16 == 0, pl16, sh)
                        eqf = jnp.where(dq == sh, 1.0, 0.0).astype(_f32)
                        for r in range(16):
                            flag = _dyng(eqf, jnp.full((16,), r, _i32))
                            row = q * 16 + r
                            for j in range(4):
                                acc[j] = (acc[j] * flag
                                          + hg[row, pl.ds(coff + j * 16, 16)])
                                pc[row, pl.ds(j * 16, 16)] = acc[j]
                        prevv[pl.ds(0, 16)] = _dyng(
                            dq, jnp.full((16,), 15, _i32))
                    for j in range(4):
                        accv[j, pl.ds(0, 16)] = acc[j]
                    pltpu.sync_copy(pc, po.at[pl.ds(off, 64)])
                    return kc

                lax.fori_loop(0, 8, chunk, 0)
                return carry

            lax.fori_loop(0, NCHB // 8, group, 0)

    return scan_kernel


def _make_pick_kernel():
    @functools.partial(
        pl.kernel,
        out_type=(jax.ShapeDtypeStruct((NNP, 128), _f32),
                  jax.ShapeDtypeStruct((NNP, 128), _f32)),
        mesh=_mesh(),
        scratch_types=[
            pltpu.VMEM((5, 64), _i32),
            pltpu.VMEM((5, 64), _i32),
            pltpu.VMEM((64, 128), _f32),
            pltpu.SemaphoreType.DMA,
        ],
    )
    def pick_kernel(p1_hbm, p2_hbm, e1_hbm, e2_hbm, o1, o2, i1, i2, hg, sem):
        c, s, wid = _widx()
        pltpu.sync_copy(e1_hbm.at[wid], i1)
        pltpu.sync_copy(e2_hbm.at[wid], i2)
        for g in range(5):
            pltpu.async_copy(p1_hbm.at[i1.at[g]], hg, sem).wait()
            pltpu.sync_copy(hg, o1.at[pl.ds(wid * NPT + g * 64, 64)])
        for g in range(5):
            pltpu.async_copy(p2_hbm.at[i2.at[g]], hg, sem).wait()
            pltpu.sync_copy(hg, o2.at[pl.ds(wid * NPT + g * 64, 64)])

    return pick_kernel


def _make_pool_kernel():
    NCHP = NNP // NW // 64

    @functools.partial(
        pl.kernel,
        out_type=jax.ShapeDtypeStruct((NNP, NGR), _f32),
        mesh=_mesh(),
        scratch_types=[
            pltpu.VMEM((NCHP * 64,), _i32),
            pltpu.VMEM((64, NGR), _f32),
            pltpu.VMEM((64, NGR), _f32),
            pltpu.VMEM((16,), _i32),
            pltpu.VMEM((8, 16), _f32),
        ],
    )
    def pool_kernel(gated_hbm, gidx_hbm, pg, dv3, gv, pc, prevv, accv):
        c, s, wid = _widx()
        iota16 = lax.iota(_i32, 16)
        shifti = jnp.maximum(iota16 - 1, 0)
        z16 = jnp.zeros((16,), _f32)
        base = wid * (NCHP * 64)
        pltpu.sync_copy(gidx_hbm.at[pl.ds(base, NCHP * 64)], dv3)
        prevv[pl.ds(0, 16)] = jnp.full((16,), -1, _i32)
        for j in range(8):
            accv[j, pl.ds(0, 16)] = z16

        def chunk(k, carry):
            off = base + k * 64
            pltpu.sync_copy(gated_hbm.at[pl.ds(off, 64)], gv)
            acc = [accv[j, pl.ds(0, 16)] for j in range(8)]
            for q in range(4):
                dq = dv3[pl.ds(k * 64 + q * 16, 16)]
                if q == 0:
                    pl16 = prevv[pl.ds(0, 16)]
                else:
                    pl16 = _dyng(dv3[pl.ds(k * 64 + (q - 1) * 16, 16)],
                                 jnp.full((16,), 15, _i32))
                sh = _dyng(dq, shifti)
                sh = jnp.where(iota16 == 0, pl16, sh)
                eqf = jnp.where(dq == sh, 1.0, 0.0).astype(_f32)
                for r in range(16):
                    flag = _dyng(eqf, jnp.full((16,), r, _i32))
                    row = q * 16 + r
                    for j in range(8):
                        acc[j] = (acc[j] * flag
                                  + gv[row, pl.ds(j * 16, 16)])
                        pc[row, pl.ds(j * 16, 16)] = acc[j]
                prevv[pl.ds(0, 16)] = _dyng(dq, jnp.full((16,), 15, _i32))
            for j in range(8):
                accv[j, pl.ds(0, 16)] = acc[j]
            pltpu.sync_copy(pc, pg.at[pl.ds(off, 64)])
            return carry

        lax.fori_loop(0, NCHP, chunk, 0)

    return pool_kernel


def _make_gpick_kernel():
    @functools.partial(
        pl.kernel,
        out_type=jax.ShapeDtypeStruct((NGR, NGR), _f32),
        mesh=_mesh(),
        scratch_types=[
            pltpu.VMEM((1, 64), _i32),
            pltpu.VMEM((64, NGR), _f32),
            pltpu.SemaphoreType.DMA,
        ],
    )
    def gpick_kernel(pg_hbm, ge_hbm, out, iv, hg, sem):
        c, s, wid = _widx()

        @pl.when(wid < 2)
        def _():
            pltpu.sync_copy(ge_hbm.at[wid], iv)
            pltpu.async_copy(pg_hbm.at[iv.at[0]], hg, sem).wait()
            pltpu.sync_copy(hg, out.at[pl.ds(wid * 64, 64)])

    return gpick_kernel


def _full(shape):
    return pl.BlockSpec(shape, lambda *_: (0,) * len(shape))


def _enc_body(nf, wn, bn, wad, wbc, xo, ado, bco):
    x = jnp.dot(nf[...], wn[...], preferred_element_type=_f32,
            precision=jax.lax.Precision.HIGHEST) + bn[...]
    xo[...] = x
    ado[...] = jnp.dot(x, wad[...], preferred_element_type=_f32,
            precision=jax.lax.Precision.HIGHEST)
    bco[...] = jnp.dot(x, wbc[...], preferred_element_type=_f32,
            precision=jax.lax.Precision.HIGHEST)


def _enc_call(nf, wn, bn, wad, wbc):
    blk = 640
    return pl.pallas_call(
        _enc_body,
        grid=(NNP // blk,),
        in_specs=[pl.BlockSpec((blk, 128), lambda i: (i, 0)),
                  _full((128, NSTATE)), _full((1, NSTATE)),
                  _full((NSTATE, 128)), _full((NSTATE, 128))],
        out_specs=[pl.BlockSpec((blk, NSTATE), lambda i: (i, 0)),
                   pl.BlockSpec((blk, 128), lambda i: (i, 0)),
                   pl.BlockSpec((blk, 128), lambda i: (i, 0))],
        out_shape=[jax.ShapeDtypeStruct((NNP, NSTATE), _f32),
                   jax.ShapeDtypeStruct((NNP, 128), _f32),
                   jax.ShapeDtypeStruct((NNP, 128), _f32)],
    )(nf, wn, bn, wad, wbc)


def _ec_body(ef, w, b, eo):
    eo[...] = jnp.dot(ef[...], w[...], preferred_element_type=_f32,
            precision=jax.lax.Precision.HIGHEST) + b[...]


def _ec_call(ef, w, b):
    blk = 10240
    return pl.pallas_call(
        _ec_body,
        grid=(NEP // blk,),
        in_specs=[pl.BlockSpec((blk, 16), lambda i: (i, 0)),
                  _full((16, 128)), _full((1, 128))],
        out_specs=pl.BlockSpec((blk, 128), lambda i: (i, 0)),
        out_shape=jax.ShapeDtypeStruct((NEP, 128), _f32),
    )(ef, w, b)


def _layer_body(s1, s2, pb1, pb2, bd1, bd2, x, cnt1, cnt2, w2, v2, b2, c2,
                u1a, u1b, u1, u2w, u2b, wad, wbc, xo, ado, bco):
    blk = s1.shape[0]
    rows = (pl.program_id(0) * blk
            + lax.broadcasted_iota(_i32, (blk, 1), 0))
    oh1 = (rows == bd1[...]).astype(_f32)
    oh2 = (rows == bd2[...]).astype(_f32)
    s1e = ((s1[...][:, :MH]
            + jnp.dot(oh1, pb1[...][:, :MH], preferred_element_type=_f32,
            precision=jax.lax.Precision.HIGHEST))
           * (cnt1[...] > 0).astype(_f32))
    s2e = ((s2[...][:, :MH]
            + jnp.dot(oh2, pb2[...][:, :MH], preferred_element_type=_f32,
            precision=jax.lax.Precision.HIGHEST))
           * (cnt2[...] > 0).astype(_f32))
    u = (jnp.dot(s1e, w2[...], preferred_element_type=_f32,
            precision=jax.lax.Precision.HIGHEST)
         + jnp.dot(s2e, v2[...], preferred_element_type=_f32,
            precision=jax.lax.Precision.HIGHEST)
         + cnt1[...] * b2[...] + cnt2[...] * c2[...])
    xv = x[...]
    t = jnp.maximum(jnp.dot(u, u1a[...], preferred_element_type=_f32,
            precision=jax.lax.Precision.HIGHEST)
                    + jnp.dot(xv, u1b[...], preferred_element_type=_f32,
            precision=jax.lax.Precision.HIGHEST)
                    + u1[...], 0.0)
    nx = xv + jnp.dot(t, u2w[...], preferred_element_type=_f32,
            precision=jax.lax.Precision.HIGHEST) + u2b[...]
    xo[...] = nx
    ado[...] = jnp.dot(nx, wad[...], preferred_element_type=_f32,
            precision=jax.lax.Precision.HIGHEST)
    bco[...] = jnp.dot(nx, wbc[...], preferred_element_type=_f32,
            precision=jax.lax.Precision.HIGHEST)


def _layer_call(s1, s2, pb1, pb2, bd1, bd2, x, cnt1, cnt2, w2, v2, b2, c2,
                u1a, u1b, u1, u2w, u2b, wad, wbc):
    blk = 640
    return pl.pallas_call(
        _layer_body,
        grid=(NNP // blk,),
        in_specs=[pl.BlockSpec((blk, 128), lambda i: (i, 0)),
                  pl.BlockSpec((blk, 128), lambda i: (i, 0)),
                  _full((NW, 128)), _full((NW, 128)),
                  _full((1, NW)), _full((1, NW)),
                  pl.BlockSpec((blk, NSTATE), lambda i: (i, 0)),
                  pl.BlockSpec((blk, 1), lambda i: (i, 0)),
                  pl.BlockSpec((blk, 1), lambda i: (i, 0)),
                  _full((MH, MH)), _full((MH, MH)),
                  _full((1, MH)), _full((1, MH)),
                  _full((MH, MH)), _full((NSTATE, MH)), _full((1, MH)),
                  _full((MH, NSTATE)), _full((1, NSTATE)),
                  _full((NSTATE, 128)), _full((NSTATE, 128))],
        out_specs=[pl.BlockSpec((blk, NSTATE), lambda i: (i, 0)),
                   pl.BlockSpec((blk, 128), lambda i: (i, 0)),
                   pl.BlockSpec((blk, 128), lambda i: (i, 0))],
        out_shape=[jax.ShapeDtypeStruct((NNP, NSTATE), _f32),
                   jax.ShapeDtypeStruct((NNP, 128), _f32),
                   jax.ShapeDtypeStruct((NNP, 128), _f32)],
    )(s1, s2, pb1, pb2, bd1, bd2, x, cnt1, cnt2, w2, v2, b2, c2,
      u1a, u1b, u1, u2w, u2b, wad, wbc)


def _gate_body(x, a1, b1, go):
    g = jnp.dot(x[...], a1[...], preferred_element_type=_f32,
            precision=jax.lax.Precision.HIGHEST) + b1[...]
    go[...] = jax.nn.sigmoid(g[:, :NGR]) * g[:, NGR:]


def _gate_call(x, a1, b1):
    blk = 640
    return pl.pallas_call(
        _gate_body,
        grid=(NNP // blk,),
        in_specs=[pl.BlockSpec((blk, NSTATE), lambda i: (i, 0)),
                  _full((NSTATE, 2 * NGR)), _full((1, 2 * NGR))],
        out_specs=pl.BlockSpec((blk, NGR), lambda i: (i, 0)),
        out_shape=jax.ShapeDtypeStruct((NNP, NGR), _f32),
    )(x, a1, b1)


def _final_body(gsr, gpb, gbd, gmask, a2, b2, pe, po, do):
    rows = lax.broadcasted_iota(_i32, (NGR, 1), 0)
    oh = (rows == gbd[...]).astype(_f32)
    gs = ((gsr[...] + jnp.dot(oh, gpb[...], preferred_element_type=_f32,
            precision=jax.lax.Precision.HIGHEST))
          * gmask[...])
    gv = jnp.dot(gs, a2[...], preferred_element_type=_f32,
            precision=jax.lax.Precision.HIGHEST) + b2[...]
    diff = (jnp.dot(pe[...], gv, preferred_element_type=_f32,
            precision=jax.lax.Precision.HIGHEST)
            - jnp.dot(po[...], gv, preferred_element_type=_f32,
            precision=jax.lax.Precision.HIGHEST))
    do[...] = jnp.sum(jnp.maximum(diff, 0.0), axis=1, keepdims=True)


def _final_call(gsr, gpb, gbd, gmask, a2, b2, pe, po):
    return pl.pallas_call(
        _final_body,
        grid=(1,),
        in_specs=[_full((NGR, NGR)), _full((NW, NGR)), _full((1, NW)),
                  _full((NGR, 1)),
                  _full((NGR, NGR)), _full((1, NGR)),
                  _full((64, NGR)), _full((64, NGR))],
        out_specs=_full((64, 1)),
        out_shape=jax.ShapeDtypeStruct((64, 1), _f32),
    )(gsr, gpb, gbd, gmask, a2, b2, pe, po)


def _csr(dest_padded):
    order = jnp.argsort(dest_padded).astype(_i32)
    dsorted = dest_padded[order].astype(_i32)
    node_end = jnp.searchsorted(
        dsorted, jnp.arange(1, NNP + 1, dtype=_i32)).astype(_i32)
    cnt = jnp.diff(jnp.concatenate(
        [jnp.zeros((1,), _i32), node_end])).astype(_f32).reshape(NNP, 1)
    endm1 = jnp.maximum(node_end - 1, 0).reshape(NW, NPT // 64, 64)
    b = jnp.arange(1, NW, dtype=_i32) * EPT - 1
    cont = dsorted[b] == dsorted[b + 1]
    bdest = jnp.where(cont, dsorted[b], NNP).astype(_i32)
    bdest = jnp.concatenate([bdest, jnp.full((1,), NNP, _i32)]).reshape(1, NW)
    return (order.reshape(NEP // 64, 64), dsorted.reshape(NEP // 64, 64),
            endm1, cnt, bdest)


def kernel(node_features, edge_features, from_idx, to_idx, graph_idx,
           n_graphs, params):
    with jax.default_matmul_precision("highest"):
        return _forward_impl(node_features, edge_features, from_idx, to_idx,
                             graph_idx, n_graphs, params)


def _forward_impl(node_features, edge_features, from_idx, to_idx, graph_idx,
                  n_graphs, params):
    p = params
    wn, bn = p['enc_node']
    we, be = p['enc_edge']
    w1, b1, w2, b2 = p['msg']
    v1, c1, v2, c2 = p['rmsg']
    uw1, ub1, uw2, ub2 = p['node_upd']
    a1w, a1b = p['agg1']
    a2w, a2b = p['agg2']

    wc = jnp.concatenate([w1[2 * NSTATE:], v1[2 * NSTATE:]], axis=1)
    wec = we @ wc
    bec = (be @ wc + jnp.concatenate([b1, c1])).reshape(1, 128)
    wad = jnp.concatenate([w1[:NSTATE], v1[NSTATE:2 * NSTATE]], axis=1)
    wbc = jnp.concatenate([w1[NSTATE:2 * NSTATE], v1[:NSTATE]], axis=1)
    u1a, u1b = uw1[:MH], uw1[MH:]

    pad_e = NEP - NE
    fpad = jnp.concatenate([from_idx, jnp.full((pad_e,), NN, _i32)])
    tpad = jnp.concatenate([to_idx, jnp.full((pad_e,), NN, _i32)])
    fidx3 = fpad.reshape(NW, NCHT, CH)
    tidx3 = tpad.reshape(NW, NCHT, CH)

    ord_t, dst_t, end_t, cnt_to, bd_t = _csr(tpad)
    ord_f, dst_f, end_f, cnt_from, bd_f = _csr(fpad)

    edge_k = _make_edge_kernel()
    scan_k = _make_scan_kernel()
    pick_k = _make_pick_kernel()
    pool_k = _make_pool_kernel()
    gpick_k = _make_gpick_kernel()

    nf_p = jnp.concatenate(
        [node_features, jnp.zeros((NNP - NN, 128), _f32)], axis=0)
    x, ad, bc = _enc_call(nf_p, wn, bn.reshape(1, -1), wad, wbc)
    ef_p = jnp.concatenate(
        [edge_features, jnp.zeros((NEP - NE, 16), _f32)], axis=0)
    e_const = _ec_call(ef_p, wec, bec)

    for _ in range(5):
        h = edge_k(ad, bc, e_const, fidx3, tidx3)
        p1, p2 = scan_k(h, ord_t, ord_f, dst_t, dst_f)
        s1, s2 = pick_k(p1, p2, end_t, end_f)
        pb1 = p1.reshape(NW, EPT, 128)[:, -1, :]
        pb2 = p2.reshape(NW, EPT, 128)[:, -1, :]
        x, ad, bc = _layer_call(
            s1, s2, pb1, pb2, bd_t, bd_f, x, cnt_to, cnt_from, w2, v2,
            b2.reshape(1, -1), c2.reshape(1, -1), u1a, u1b,
            ub1.reshape(1, -1), uw2, ub2.reshape(1, -1), wad, wbc)

    gated = _gate_call(x, a1w, a1b.reshape(1, -1))
    gidx = jnp.minimum(graph_idx, n_graphs - 1).astype(_i32)
    gidx_p = jnp.concatenate([gidx, jnp.full((NNP - NN,), NGR, _i32)])
    gend = jnp.searchsorted(
        gidx_p, jnp.arange(1, NGR + 1, dtype=_i32)).astype(_i32)
    gcnt = jnp.diff(jnp.concatenate([jnp.zeros((1,), _i32), gend]))
    gmask = (gcnt > 0).astype(_f32).reshape(NGR, 1)
    gendm1 = jnp.maximum(gend - 1, 0).reshape(2, 1, 64)
    gendm1 = jnp.concatenate(
        [gendm1, jnp.zeros((NW - 2, 1, 64), _i32)], axis=0)
    bg = jnp.arange(1, NW, dtype=_i32) * NPT - 1
    gsd = gidx_p
    gcont = gsd[bg] == gsd[bg + 1]
    gbd = jnp.where(gcont, gsd[bg], NGR).astype(_i32)
    gbd = jnp.concatenate([gbd, jnp.full((1,), NGR, _i32)]).reshape(1, NW)

    pg = pool_k(gated, gidx_p)
    gsr = gpick_k(pg, gendm1)
    gpb = pg.reshape(NW, NPT, NGR)[:, -1, :]

    pe = jnp.zeros((64, NGR), _f32).at[jnp.arange(64), 2 * jnp.arange(64)].set(1.0)
    po = jnp.zeros((64, NGR), _f32).at[jnp.arange(64), 2 * jnp.arange(64) + 1].set(1.0)
    d = _final_call(gsr, gpb, gbd, gmask, a2w, a2b.reshape(1, -1), pe, po)
    return d.reshape(64)
